# baseline (speedup 1.0000x reference)
import functools

import jax
import jax.numpy as jnp
from jax import lax
from jax.experimental import pallas as pl
from jax.experimental.pallas import tpu as pltpu
from jax.experimental.pallas import tpu_sc as plsc

B, S, D = 4, 8192, 1024
NC, NS, L = 2, 16, 16
NW = NC * NS
ROWS_W = S // NW
R = 32
NT = ROWS_W // R
CH = R * D
UNROLL = 8


def _sc_add(x_flat, pos_flat):
    mesh = plsc.VectorSubcoreMesh(core_axis_name="c", subcore_axis_name="s")

    @functools.partial(
        pl.kernel,
        mesh=mesh,
        out_type=jax.ShapeDtypeStruct((B * S * D,), jnp.float32),
        scratch_types=[
            pltpu.VMEM((CH,), jnp.float32),
            pltpu.VMEM((CH,), jnp.float32),
        ],
    )
    def k(x_hbm, pos_hbm, out_hbm, pos_v, x_v):
        wid = lax.axis_index("s") * NC + lax.axis_index("c")
        base = wid * (ROWS_W * D)

        def t_body(t, carry):
            off = base + t * CH
            pltpu.sync_copy(pos_hbm.at[pl.ds(off, CH)], pos_v)

            def b_body(b, carry2):
                boff = b * (S * D) + off
                pltpu.sync_copy(x_hbm.at[pl.ds(boff, CH)], x_v)

                def j_body(j, carry3):
                    o = j * (L * UNROLL)
                    for u in range(UNROLL):
                        sl = pl.ds(o + u * L, L)
                        x_v[sl] = x_v[sl] + pos_v[sl]
                    return carry3

                lax.fori_loop(0, CH // (L * UNROLL), j_body, 0, unroll=False)
                pltpu.sync_copy(x_v, out_hbm.at[pl.ds(boff, CH)])
                return carry2

            lax.fori_loop(0, B, b_body, 0, unroll=False)
            return carry

        lax.fori_loop(0, NT, t_body, 0, unroll=False)

    return k(x_flat, pos_flat)


def kernel(x, pos_embedding):
    out = _sc_add(x.reshape(B * S * D), pos_embedding.reshape(S * D))
    return out.reshape(B, S, D)


# speedup vs baseline: 1.2206x; 1.2206x over previous
import functools

import jax
import jax.numpy as jnp
from jax import lax
from jax.experimental import pallas as pl
from jax.experimental.pallas import tpu as pltpu
from jax.experimental.pallas import tpu_sc as plsc

B, S, D = 4, 8192, 1024
NC, NS, L = 2, 16, 16
NW = NC * NS
ROWS_W = S // NW
R = 8
CH = R * D
NT = ROWS_W // R
U = 4


def _sc_add(x_flat, pos_flat):
    mesh = plsc.VectorSubcoreMesh(core_axis_name="c", subcore_axis_name="s")

    @functools.partial(
        pl.kernel,
        mesh=mesh,
        out_type=jax.ShapeDtypeStruct((B * S * D,), jnp.float32),
        scratch_types=[
            pltpu.VMEM((B * CH,), jnp.float32),
            pltpu.VMEM((B * CH,), jnp.float32),
            pltpu.VMEM((B * CH,), jnp.float32),
            pltpu.VMEM((CH,), jnp.float32),
            pltpu.VMEM((CH,), jnp.float32),
            pltpu.VMEM((CH,), jnp.float32),
            pltpu.SemaphoreType.DMA,
            pltpu.SemaphoreType.DMA,
            pltpu.SemaphoreType.DMA,
            pltpu.SemaphoreType.DMA,
            pltpu.SemaphoreType.DMA,
            pltpu.SemaphoreType.DMA,
        ],
    )
    def k(x_hbm, pos_hbm, out_hbm, x0, x1, x2, p0, p1, p2,
          si0, si1, si2, so0, so1, so2):
        xs, ps = [x0, x1, x2], [p0, p1, p2]
        sin, sout = [si0, si1, si2], [so0, so1, so2]
        wid = lax.axis_index("s") * NC + lax.axis_index("c")
        rbase = wid * ROWS_W

        in_handles = {}
        out_handles = {}

        def issue_in(i):
            s = i % 3
            off = (rbase + i * R) * D
            hs = [pltpu.async_copy(pos_hbm.at[pl.ds(off, CH)], ps[s], sin[s])]
            for b in range(B):
                hs.append(pltpu.async_copy(
                    x_hbm.at[pl.ds(b * (S * D) + off, CH)],
                    xs[s].at[pl.ds(b * CH, CH)], sin[s]))
            in_handles[s] = hs

        def issue_out(i):
            s = i % 3
            off = (rbase + i * R) * D
            hs = []
            for b in range(B):
                hs.append(pltpu.async_copy(
                    xs[s].at[pl.ds(b * CH, CH)],
                    out_hbm.at[pl.ds(b * (S * D) + off, CH)], sout[s]))
            out_handles[s] = hs

        def drain(handles, s):
            for h in handles.pop(s, ()):
                h.wait()

        def compute(s):
            xv, pv_ref = xs[s], ps[s]

            def body(j, carry):
                o0 = j * (L * U)
                for u in range(U):
                    o = o0 + u * L
                    pv = pv_ref[pl.ds(o, L)]
                    for b in range(B):
                        sl = pl.ds(b * CH + o, L)
                        xv[sl] = xv[sl] + pv
                return carry

            lax.fori_loop(0, CH // (L * U), body, 0, unroll=False)

        issue_in(0)
        for i in range(NT):
            if i + 1 < NT:
                drain(out_handles, (i + 1) % 3)
                issue_in(i + 1)
            drain(in_handles, i % 3)
            compute(i % 3)
            issue_out(i)
        for s in range(3):
            drain(out_handles, s)

    return k(x_flat, pos_flat)


def kernel(x, pos_embedding):
    out = _sc_add(x.reshape(B * S * D), pos_embedding.reshape(S * D))
    return out.reshape(B, S, D)


# speedup vs baseline: 3.4838x; 2.8541x over previous
import functools

import jax
import jax.numpy as jnp
from jax import lax
from jax.experimental import pallas as pl
from jax.experimental.pallas import tpu as pltpu
from jax.experimental.pallas import tpu_sc as plsc

B, S, D = 4, 8192, 1024
NC, NS, L = 2, 16, 16
NW = NC * NS
ROWS_W = S // NW
R = 8
NT = ROWS_W // R


def _sc_add(x, pos):
    mesh = plsc.VectorSubcoreMesh(core_axis_name="c", subcore_axis_name="s")

    @functools.partial(
        pl.kernel,
        mesh=mesh,
        out_type=jax.ShapeDtypeStruct((B, S, D), jnp.float32),
        scratch_types=[
            pltpu.VMEM((B, R, D), jnp.float32),
            pltpu.VMEM((B, R, D), jnp.float32),
            pltpu.VMEM((B, R, D), jnp.float32),
            pltpu.VMEM((R, D), jnp.float32),
            pltpu.VMEM((R, D), jnp.float32),
            pltpu.VMEM((R, D), jnp.float32),
            pltpu.SemaphoreType.DMA,
            pltpu.SemaphoreType.DMA,
            pltpu.SemaphoreType.DMA,
            pltpu.SemaphoreType.DMA,
            pltpu.SemaphoreType.DMA,
            pltpu.SemaphoreType.DMA,
        ],
    )
    def k(x_hbm, pos_hbm, out_hbm, x0, x1, x2, p0, p1, p2,
          si0, si1, si2, so0, so1, so2):
        xs, ps = [x0, x1, x2], [p0, p1, p2]
        sin, sout = [si0, si1, si2], [so0, so1, so2]
        wid = lax.axis_index("s") * NC + lax.axis_index("c")
        rbase = wid * ROWS_W

        in_handles = {}
        out_handles = {}

        def issue_in(i):
            s = i % 3
            r0 = rbase + i * R
            hs = [pltpu.async_copy(pos_hbm.at[pl.ds(r0, R)], ps[s], sin[s])]
            for b in range(B):
                hs.append(pltpu.async_copy(
                    x_hbm.at[b, pl.ds(r0, R)], xs[s].at[b], sin[s]))
            in_handles[s] = hs

        def issue_out(i):
            s = i % 3
            r0 = rbase + i * R
            hs = []
            for b in range(B):
                hs.append(pltpu.async_copy(
                    xs[s].at[b], out_hbm.at[b, pl.ds(r0, R)], sout[s]))
            out_handles[s] = hs

        def drain(handles, s):
            for h in handles.pop(s, ()):
                h.wait()

        def compute(s):
            xv, pv_ref = xs[s], ps[s]

            def body(j, carry):
                o = j * L
                for r in range(R):
                    pv = pv_ref[r, pl.ds(o, L)]
                    for b in range(B):
                        xv[b, r, pl.ds(o, L)] = xv[b, r, pl.ds(o, L)] + pv
                return carry

            lax.fori_loop(0, D // L, body, 0, unroll=False)

        issue_in(0)
        for i in range(NT):
            if i + 1 < NT:
                drain(out_handles, (i + 1) % 3)
                issue_in(i + 1)
            drain(in_handles, i % 3)
            compute(i % 3)
            issue_out(i)
        for s in range(3):
            drain(out_handles, s)

    return k(x, pos)


def kernel(x, pos_embedding):
    return _sc_add(x, pos_embedding)


# speedup vs baseline: 3.5445x; 1.0174x over previous
import functools

import jax
import jax.numpy as jnp
from jax import lax
from jax.experimental import pallas as pl
from jax.experimental.pallas import tpu as pltpu
from jax.experimental.pallas import tpu_sc as plsc

B, S, D = 4, 8192, 1024
NC, NS, L = 2, 16, 16
NW = NC * NS
ROWS_W = S // NW
R = 8
NT = ROWS_W // R


def _sc_add(x, pos):
    mesh = plsc.VectorSubcoreMesh(core_axis_name="c", subcore_axis_name="s")

    @functools.partial(
        pl.kernel,
        mesh=mesh,
        out_type=jax.ShapeDtypeStruct((B, S, D), jnp.float32),
        scratch_types=[
            pltpu.VMEM((B, R, D), jnp.float32),
            pltpu.VMEM((B, R, D), jnp.float32),
            pltpu.VMEM((B, R, D), jnp.float32),
            pltpu.VMEM((R, D), jnp.float32),
            pltpu.VMEM((R, D), jnp.float32),
            pltpu.VMEM((R, D), jnp.float32),
            pltpu.SemaphoreType.DMA,
            pltpu.SemaphoreType.DMA,
            pltpu.SemaphoreType.DMA,
            pltpu.SemaphoreType.DMA,
            pltpu.SemaphoreType.DMA,
            pltpu.SemaphoreType.DMA,
        ],
    )
    def k(x_hbm, pos_hbm, out_hbm, x0, x1, x2, p0, p1, p2,
          si0, si1, si2, so0, so1, so2):
        xs, ps = [x0, x1, x2], [p0, p1, p2]
        sin, sout = [si0, si1, si2], [so0, so1, so2]
        wid = lax.axis_index("s") * NC + lax.axis_index("c")
        rbase = wid * ROWS_W

        in_handles = {}
        out_handles = {}

        def issue_in(i):
            s = i % 3
            r0 = rbase + i * R
            hs = [pltpu.async_copy(pos_hbm.at[pl.ds(r0, R)], ps[s], sin[s])]
            for b in range(B):
                hs.append(pltpu.async_copy(
                    x_hbm.at[b, pl.ds(r0, R)], xs[s].at[b], sin[s]))
            in_handles[s] = hs

        def issue_out(i):
            s = i % 3
            r0 = rbase + i * R
            hs = []
            for b in range(B):
                hs.append(pltpu.async_copy(
                    xs[s].at[b], out_hbm.at[b, pl.ds(r0, R)], sout[s]))
            out_handles[s] = hs

        def drain(handles, s):
            for h in handles.pop(s, ()):
                h.wait()

        def compute(s):
            xv, pv_ref = xs[s], ps[s]

            @plsc.parallel_loop(0, D // L, step=1)
            def body(j):
                o = j * L
                for r in range(R):
                    pv = pv_ref[r, pl.ds(o, L)]
                    for b in range(B):
                        xv[b, r, pl.ds(o, L)] = xv[b, r, pl.ds(o, L)] + pv

        issue_in(0)
        for i in range(NT):
            if i + 1 < NT:
                drain(out_handles, (i + 1) % 3)
                issue_in(i + 1)
            drain(in_handles, i % 3)
            compute(i % 3)
            issue_out(i)
        for s in range(3):
            drain(out_handles, s)

    return k(x, pos)


def kernel(x, pos_embedding):
    return _sc_add(x, pos_embedding)


# speedup vs baseline: 3.5765x; 1.0090x over previous
import functools

import jax
import jax.numpy as jnp
from jax import lax
from jax.experimental import pallas as pl
from jax.experimental.pallas import tpu as pltpu
from jax.experimental.pallas import tpu_sc as plsc

B, S, D = 4, 8192, 1024
NC, NS, L = 2, 16, 16
NW = NC * NS
ROWS_W = S // NW
R = 8
NT = ROWS_W // R


def _sc_add(x, pos):
    mesh = plsc.VectorSubcoreMesh(core_axis_name="c", subcore_axis_name="s")

    @functools.partial(
        pl.kernel,
        mesh=mesh,
        out_type=jax.ShapeDtypeStruct((B, S, D), jnp.float32),
        scratch_types=[
            pltpu.VMEM((B, R, D), jnp.float32),
            pltpu.VMEM((B, R, D), jnp.float32),
            pltpu.VMEM((B, R, D), jnp.float32),
            pltpu.VMEM((R, D), jnp.float32),
            pltpu.VMEM((R, D), jnp.float32),
            pltpu.VMEM((R, D), jnp.float32),
            pltpu.SemaphoreType.DMA,
            pltpu.SemaphoreType.DMA,
            pltpu.SemaphoreType.DMA,
            pltpu.SemaphoreType.DMA,
            pltpu.SemaphoreType.DMA,
            pltpu.SemaphoreType.DMA,
        ],
    )
    def k(x_hbm, pos_hbm, out_hbm, x0, x1, x2, p0, p1, p2,
          si0, si1, si2, so0, so1, so2):
        xs, ps = [x0, x1, x2], [p0, p1, p2]
        sin, sout = [si0, si1, si2], [so0, so1, so2]
        wid = lax.axis_index("s") * NC + lax.axis_index("c")
        rbase = wid * ROWS_W

        in_handles = {}
        out_handles = {}

        def issue_in(i):
            s = i % 3
            r0 = rbase + i * R
            in_handles[s] = [
                pltpu.async_copy(pos_hbm.at[pl.ds(r0, R)], ps[s], sin[s]),
                pltpu.async_copy(x_hbm.at[:, pl.ds(r0, R)], xs[s], sin[s]),
            ]

        def issue_out(i):
            s = i % 3
            r0 = rbase + i * R
            out_handles[s] = [
                pltpu.async_copy(xs[s], out_hbm.at[:, pl.ds(r0, R)], sout[s]),
            ]

        def drain(handles, s):
            for h in handles.pop(s, ()):
                h.wait()

        def compute(s):
            xv, pv_ref = xs[s], ps[s]

            @plsc.parallel_loop(0, D // L, step=1)
            def body(j):
                o = j * L
                for r in range(R):
                    pv = pv_ref[r, pl.ds(o, L)]
                    for b in range(B):
                        xv[b, r, pl.ds(o, L)] = xv[b, r, pl.ds(o, L)] + pv

        issue_in(0)
        for i in range(NT):
            if i + 1 < NT:
                drain(out_handles, (i + 1) % 3)
                issue_in(i + 1)
            drain(in_handles, i % 3)
            compute(i % 3)
            issue_out(i)
        for s in range(3):
            drain(out_handles, s)

    return k(x, pos)


def kernel(x, pos_embedding):
    return _sc_add(x, pos_embedding)


# speedup vs baseline: 3.6203x; 1.0122x over previous
import functools

import jax
import jax.numpy as jnp
from jax import lax
from jax.experimental import pallas as pl
from jax.experimental.pallas import tpu as pltpu
from jax.experimental.pallas import tpu_sc as plsc

B, S, D = 4, 8192, 1024
NC, NS, L = 2, 16, 16
NW = NC * NS
ROWS_W = S // NW
R = 8
NT = ROWS_W // R


def _sc_add(x, pos):
    mesh = plsc.VectorSubcoreMesh(core_axis_name="c", subcore_axis_name="s")

    @functools.partial(
        pl.kernel,
        mesh=mesh,
        out_type=jax.ShapeDtypeStruct((B, S, D), jnp.float32),
        scratch_types=[
            pltpu.VMEM((B, R, D), jnp.float32),
            pltpu.VMEM((B, R, D), jnp.float32),
            pltpu.VMEM((B, R, D), jnp.float32),
            pltpu.VMEM((R, D), jnp.float32),
            pltpu.VMEM((R, D), jnp.float32),
            pltpu.VMEM((R, D), jnp.float32),
            pltpu.SemaphoreType.DMA,
            pltpu.SemaphoreType.DMA,
            pltpu.SemaphoreType.DMA,
            pltpu.SemaphoreType.DMA,
            pltpu.SemaphoreType.DMA,
            pltpu.SemaphoreType.DMA,
        ],
    )
    def k(x_hbm, pos_hbm, out_hbm, x0, x1, x2, p0, p1, p2,
          si0, si1, si2, so0, so1, so2):
        xs, ps = [x0, x1, x2], [p0, p1, p2]
        sin, sout = [si0, si1, si2], [so0, so1, so2]
        wid = lax.axis_index("s") * NC + lax.axis_index("c")

        in_handles = {}
        out_handles = {}

        def issue_in(i):
            s = i % 3
            r0 = (i * NW + wid) * R
            in_handles[s] = [
                pltpu.async_copy(pos_hbm.at[pl.ds(r0, R)], ps[s], sin[s]),
                pltpu.async_copy(x_hbm.at[:, pl.ds(r0, R)], xs[s], sin[s]),
            ]

        def issue_out(i):
            s = i % 3
            r0 = (i * NW + wid) * R
            out_handles[s] = [
                pltpu.async_copy(xs[s], out_hbm.at[:, pl.ds(r0, R)], sout[s]),
            ]

        def drain(handles, s):
            for h in handles.pop(s, ()):
                h.wait()

        def compute(s):
            xv, pv_ref = xs[s], ps[s]

            @plsc.parallel_loop(0, D // L, step=1)
            def body(j):
                o = j * L
                for r in range(R):
                    pv = pv_ref[r, pl.ds(o, L)]
                    for b in range(B):
                        xv[b, r, pl.ds(o, L)] = xv[b, r, pl.ds(o, L)] + pv

        issue_in(0)
        for i in range(NT):
            if i + 1 < NT:
                drain(out_handles, (i + 1) % 3)
                issue_in(i + 1)
            drain(in_handles, i % 3)
            compute(i % 3)
            issue_out(i)
        for s in range(3):
            drain(out_handles, s)

    return k(x, pos)


def kernel(x, pos_embedding):
    return _sc_add(x, pos_embedding)


# speedup vs baseline: 3.6214x; 1.0003x over previous
import functools

import jax
import jax.numpy as jnp
from jax import lax
from jax.experimental import pallas as pl
from jax.experimental.pallas import tpu as pltpu
from jax.experimental.pallas import tpu_sc as plsc

B, S, D = 4, 8192, 1024
NC, NS, L = 2, 16, 16
NW = NC * NS
R = 4
NT = S // (NW * R)
SETS = 6
AHEAD = 2


def _sc_add(x, pos):
    mesh = plsc.VectorSubcoreMesh(core_axis_name="c", subcore_axis_name="s")

    @functools.partial(
        pl.kernel,
        mesh=mesh,
        out_type=jax.ShapeDtypeStruct((B, S, D), jnp.float32),
        scratch_types=(
            [pltpu.VMEM((B, R, D), jnp.float32)] * SETS
            + [pltpu.VMEM((R, D), jnp.float32)] * SETS
            + [pltpu.SemaphoreType.DMA] * (2 * SETS)
        ),
    )
    def k(x_hbm, pos_hbm, out_hbm, *scr):
        xs = scr[:SETS]
        ps = scr[SETS:2 * SETS]
        sin = scr[2 * SETS:3 * SETS]
        sout = scr[3 * SETS:4 * SETS]
        wid = lax.axis_index("s") * NC + lax.axis_index("c")

        in_handles = {}
        out_handles = {}

        def issue_in(i):
            s = i % SETS
            r0 = (i * NW + wid) * R
            in_handles[s] = [
                pltpu.async_copy(pos_hbm.at[pl.ds(r0, R)], ps[s], sin[s]),
                pltpu.async_copy(x_hbm.at[:, pl.ds(r0, R)], xs[s], sin[s]),
            ]

        def issue_out(i):
            s = i % SETS
            r0 = (i * NW + wid) * R
            out_handles[s] = [
                pltpu.async_copy(xs[s], out_hbm.at[:, pl.ds(r0, R)], sout[s]),
            ]

        def drain(handles, s):
            for h in handles.pop(s, ()):
                h.wait()

        def compute(s):
            xv, pv_ref = xs[s], ps[s]

            @plsc.parallel_loop(0, D // L, step=1)
            def body(j):
                o = j * L
                for r in range(R):
                    pv = pv_ref[r, pl.ds(o, L)]
                    for b in range(B):
                        xv[b, r, pl.ds(o, L)] = xv[b, r, pl.ds(o, L)] + pv

        for j in range(AHEAD):
            issue_in(j)
        for i in range(NT):
            nxt = i + AHEAD
            if nxt < NT:
                drain(out_handles, nxt % SETS)
                issue_in(nxt)
            drain(in_handles, i % SETS)
            compute(i % SETS)
            issue_out(i)
        for s in range(SETS):
            drain(out_handles, s)

    return k(x, pos)


def kernel(x, pos_embedding):
    return _sc_add(x, pos_embedding)


# speedup vs baseline: 3.9438x; 1.0890x over previous
import functools

import jax
import jax.numpy as jnp
from jax import lax
from jax.experimental import pallas as pl
from jax.experimental.pallas import tpu as pltpu
from jax.experimental.pallas import tpu_sc as plsc

B, S, D = 4, 8192, 1024
NC, NS, L = 2, 16, 16
NW = NC * NS
R = 4
NT = S // (NW * R)
SETS = 4
AHEAD = 2
G = NT // SETS


def _sc_add(x, pos):
    mesh = plsc.VectorSubcoreMesh(core_axis_name="c", subcore_axis_name="s")

    @functools.partial(
        pl.kernel,
        mesh=mesh,
        out_type=jax.ShapeDtypeStruct((B, S, D), jnp.float32),
        scratch_types=(
            [pltpu.VMEM((B, R, D), jnp.float32)] * SETS
            + [pltpu.VMEM((R, D), jnp.float32)] * SETS
            + [pltpu.SemaphoreType.DMA] * (2 * SETS)
        ),
    )
    def k(x_hbm, pos_hbm, out_hbm, *scr):
        xs = scr[:SETS]
        ps = scr[SETS:2 * SETS]
        sin = scr[2 * SETS:3 * SETS]
        sout = scr[3 * SETS:4 * SETS]
        wid = lax.axis_index("s") * NC + lax.axis_index("c")

        def issue_in(j, s):
            r0 = (j * NW + wid) * R
            pltpu.async_copy(pos_hbm.at[pl.ds(r0, R)], ps[s], sin[s])
            pltpu.async_copy(x_hbm.at[:, pl.ds(r0, R)], xs[s], sin[s])

        def wait_in(s):
            pltpu.make_async_copy(pos_hbm.at[pl.ds(0, R)], ps[s], sin[s]).wait()
            pltpu.make_async_copy(x_hbm.at[:, pl.ds(0, R)], xs[s], sin[s]).wait()

        def issue_out(j, s):
            r0 = (j * NW + wid) * R
            pltpu.async_copy(xs[s], out_hbm.at[:, pl.ds(r0, R)], sout[s])

        def wait_out(s):
            pltpu.make_async_copy(x_hbm.at[:, pl.ds(0, R)], xs[s], sout[s]).wait()

        def compute(s):
            xv, pv_ref = xs[s], ps[s]

            @plsc.parallel_loop(0, D // L, step=1)
            def body(j):
                o = j * L
                for r in range(R):
                    pv = pv_ref[r, pl.ds(o, L)]
                    for b in range(B):
                        xv[b, r, pl.ds(o, L)] = xv[b, r, pl.ds(o, L)] + pv

        for j in range(AHEAD):
            issue_in(j, j % SETS)

        def g_body(g, carry):
            i0 = g * SETS
            for s in range(SETS):
                i = i0 + s
                nxt = i + AHEAD
                t = (s + AHEAD) % SETS

                @pl.when(jnp.logical_and(nxt >= SETS, nxt < NT))
                def _():
                    wait_out(t)

                @pl.when(nxt < NT)
                def _():
                    issue_in(nxt, t)

                wait_in(s)
                compute(s)
                issue_out(i, s)
            return carry

        lax.fori_loop(0, G, g_body, 0, unroll=False)
        for s in range(SETS):
            wait_out(s)

    return k(x, pos)


def kernel(x, pos_embedding):
    return _sc_add(x, pos_embedding)


# speedup vs baseline: 3.9525x; 1.0022x over previous
import functools

import jax
import jax.numpy as jnp
from jax import lax
from jax.experimental import pallas as pl
from jax.experimental.pallas import tpu as pltpu
from jax.experimental.pallas import tpu_sc as plsc

B, S, D = 4, 8192, 1024
NC, NS, L = 2, 16, 16
NW = NC * NS
R = 4
NT = S // (NW * R)
SETS = 4
AHEAD = 2
G = NT // SETS


def _sc_add(x, pos):
    mesh = plsc.VectorSubcoreMesh(core_axis_name="c", subcore_axis_name="s")

    @functools.partial(
        pl.kernel,
        mesh=mesh,
        out_type=jax.ShapeDtypeStruct((B, S, D), jnp.float32),
        scratch_types=(
            [pltpu.VMEM((B, R, D), jnp.float32)] * SETS
            + [pltpu.VMEM((R, D), jnp.float32)] * SETS
            + [pltpu.SemaphoreType.DMA] * (2 * SETS)
        ),
    )
    def k(x_hbm, pos_hbm, out_hbm, *scr):
        xs = scr[:SETS]
        ps = scr[SETS:2 * SETS]
        sin = scr[2 * SETS:3 * SETS]
        sout = scr[3 * SETS:4 * SETS]
        wid = lax.axis_index("s") * NC + lax.axis_index("c")

        def issue_in(j, s):
            r0 = (j * NW + wid) * R
            pltpu.async_copy(pos_hbm.at[pl.ds(r0, R)], ps[s], sin[s])
            pltpu.async_copy(x_hbm.at[:, pl.ds(r0, R)], xs[s], sin[s])

        def wait_in(s):
            pltpu.make_async_copy(pos_hbm.at[pl.ds(0, R)], ps[s], sin[s]).wait()
            pltpu.make_async_copy(x_hbm.at[:, pl.ds(0, R)], xs[s], sin[s]).wait()

        def issue_out(j, s):
            r0 = (j * NW + wid) * R
            pltpu.async_copy(xs[s], out_hbm.at[:, pl.ds(r0, R)], sout[s])

        def wait_out(s):
            pltpu.make_async_copy(x_hbm.at[:, pl.ds(0, R)], xs[s], sout[s]).wait()

        def compute(s):
            xv, pv_ref = xs[s], ps[s]

            @plsc.parallel_loop(0, D // L, step=1)
            def body(j):
                o = j * L
                for r in range(R):
                    pv = pv_ref[r, pl.ds(o, L)]
                    for b in range(B):
                        plsc.addupdate(xv.at[b, r, pl.ds(o, L)], pv)

        for j in range(AHEAD):
            issue_in(j, j % SETS)

        def g_body(g, carry):
            i0 = g * SETS
            for s in range(SETS):
                i = i0 + s
                nxt = i + AHEAD
                t = (s + AHEAD) % SETS

                @pl.when(jnp.logical_and(nxt >= SETS, nxt < NT))
                def _():
                    wait_out(t)

                @pl.when(nxt < NT)
                def _():
                    issue_in(nxt, t)

                wait_in(s)
                compute(s)
                issue_out(i, s)
            return carry

        lax.fori_loop(0, G, g_body, 0, unroll=False)
        for s in range(SETS):
            wait_out(s)

    return k(x, pos)


def kernel(x, pos_embedding):
    return _sc_add(x, pos_embedding)
